# baseline (speedup 1.0000x reference)
import functools

import jax
import jax.numpy as jnp
from jax import lax
from jax.experimental import pallas as pl
from jax.experimental.pallas import tpu as pltpu
from jax.experimental.pallas import tpu_sc as plsc

_FACTOR = 3
_IGNORE = -100
_N = 262144
_C = 21
_BLK = 131072
_G = _N // _BLK

_NW = 32
_RPW = _N // _NW


def _ce_body(pred_ref, label_ref, lse_ref, picked_ref):
    x = pred_ref[...]
    lab = label_ref[0, 0, :]
    m = jnp.max(x)
    e = jnp.exp(x - m)
    s = jnp.sum(e, axis=0)
    lse = jnp.log(s) + m
    cls = lax.broadcasted_iota(jnp.int32, x.shape, 0)
    picked = jnp.sum(jnp.where(cls == lab[None, :], x, 0.0), axis=0)
    i = pl.program_id(0)
    lse_ref[0, i] = jnp.sum(lse)
    picked_ref[0, i] = jnp.sum(picked)


def _ce_pass(pred_t, label):
    label3 = label.reshape(_G, 1, _BLK)
    out = pl.pallas_call(
        _ce_body,
        grid=(_G,),
        in_specs=[
            pl.BlockSpec((_C, _BLK), lambda i: (0, i)),
            pl.BlockSpec((1, 1, _BLK), lambda i: (i, 0, 0)),
        ],
        out_specs=[
            pl.BlockSpec((1, _G), lambda i: (0, 0), memory_space=pltpu.SMEM),
            pl.BlockSpec((1, _G), lambda i: (0, 0), memory_space=pltpu.SMEM),
        ],
        out_shape=[
            jax.ShapeDtypeStruct((1, _G), jnp.float32),
            jax.ShapeDtypeStruct((1, _G), jnp.float32),
        ],
    )(pred_t, label3)
    return jnp.sum(out[0]), jnp.sum(out[1])


@functools.partial(
    pl.kernel,
    mesh=plsc.VectorSubcoreMesh(core_axis_name="c", subcore_axis_name="s"),
    out_type=jax.ShapeDtypeStruct((_NW, 16), jnp.float32),
    scratch_types=[
        pltpu.VMEM((_RPW,), jnp.int32),
        pltpu.VMEM((16,), jnp.float32),
    ],
    compiler_params=pltpu.CompilerParams(
        use_tc_tiling_on_sc=False, needs_layout_passes=False
    ),
)
def _sc_pos_count(label_hbm, out_hbm, lab_v, res_v):
    wid = lax.axis_index("s") * 2 + lax.axis_index("c")
    pltpu.sync_copy(label_hbm.at[pl.ds(wid * _RPW, _RPW)], lab_v)

    def body(k, p):
        labv = lab_v[pl.ds(k * 16, 16)]
        return p + jnp.where(labv != 0, 1.0, 0.0)

    pos = lax.fori_loop(0, _RPW // 16, body, jnp.zeros((16,), jnp.float32))
    res_v[...] = pos
    pltpu.sync_copy(res_v, out_hbm.at[wid])


def _rare_ohem_branch(ops):
    pred, label, pos_num, neg_count, neg_sum = ops
    pred_value = jnp.max(pred[:, 1:], axis=1)
    is_neg = label == 0
    padded = jnp.where(is_neg, -pred_value, jnp.inf)
    sorted_neg_score = jnp.sort(padded)
    raw_idx = neg_sum - 1
    idx = jnp.where(raw_idx >= 0, raw_idx, neg_count + raw_idx)
    idx = jnp.clip(idx, 0, padded.shape[0] - 1)
    threshold = -sorted_neg_score[idx]
    mask = (pred_value >= threshold) | (label != 0)
    masked_label = jnp.where(mask, label, _IGNORE)
    logp = jax.nn.log_softmax(pred, axis=1)
    valid = masked_label != _IGNORE
    safe = jnp.where(valid, masked_label, 0)
    nll = -jnp.take_along_axis(logp, safe[:, None], axis=1)[:, 0]
    denom = jnp.maximum(jnp.sum(valid), 1).astype(pred.dtype)
    return jnp.sum(jnp.where(valid, nll, 0.0)) / denom


def kernel(pred, label):
    pos_parts = _sc_pos_count(label)
    pred_t = pred.T
    sum_lse, sum_picked = _ce_pass(pred_t, label)
    pos_num = jnp.sum(pos_parts).astype(jnp.int32)
    neg_count = _N - pos_num
    neg_sum = pos_num * _FACTOR
    common = (sum_lse - sum_picked) / jnp.float32(_N)
    return lax.cond(
        neg_count > neg_sum,
        _rare_ohem_branch,
        lambda ops: common,
        (pred, label, pos_num, neg_count, neg_sum),
    )


# speedup vs baseline: 1.0321x; 1.0321x over previous
import functools

import jax
import jax.numpy as jnp
from jax import lax
from jax.experimental import pallas as pl
from jax.experimental.pallas import tpu as pltpu
from jax.experimental.pallas import tpu_sc as plsc

_FACTOR = 3
_IGNORE = -100
_N = 262144
_C = 21
_BLK = 65536
_G = _N // _BLK

_NW = 32
_RPW = _N // _NW


def _ce_body(pred_ref, label_ref, lse_ref, picked_ref):
    x = pred_ref[...]
    lab = label_ref[0, 0, :]
    m = jnp.max(x)
    e = jnp.exp(x - m)
    s = jnp.sum(e, axis=0)
    lse = jnp.log(s) + m
    cls = lax.broadcasted_iota(jnp.int32, x.shape, 0)
    picked = jnp.sum(jnp.where(cls == lab[None, :], x, 0.0), axis=0)
    i = pl.program_id(0)
    lse_ref[0, i] = jnp.sum(lse)
    picked_ref[0, i] = jnp.sum(picked)


def _ce_pass(pred_t, label):
    label3 = label.reshape(_G, 1, _BLK)
    out = pl.pallas_call(
        _ce_body,
        grid=(_G,),
        in_specs=[
            pl.BlockSpec((_C, _BLK), lambda i: (0, i)),
            pl.BlockSpec((1, 1, _BLK), lambda i: (i, 0, 0)),
        ],
        out_specs=[
            pl.BlockSpec((1, _G), lambda i: (0, 0), memory_space=pltpu.SMEM),
            pl.BlockSpec((1, _G), lambda i: (0, 0), memory_space=pltpu.SMEM),
        ],
        out_shape=[
            jax.ShapeDtypeStruct((1, _G), jnp.float32),
            jax.ShapeDtypeStruct((1, _G), jnp.float32),
        ],
    )(pred_t, label3)
    return jnp.sum(out[0]), jnp.sum(out[1])


def _sc_pos_count_body(label_hbm, out_hbm, lab_v, res_v):
    wid = lax.axis_index("s") * 2 + lax.axis_index("c")
    pltpu.sync_copy(label_hbm.at[pl.ds(wid * _RPW, _RPW)], lab_v)

    def body(k, p):
        labv = lab_v[pl.ds(k * 16, 16)]
        return p + jnp.where(labv != 0, 1.0, 0.0)

    pos = lax.fori_loop(0, _RPW // 16, body, jnp.zeros((16,), jnp.float32))
    res_v[...] = pos
    pltpu.sync_copy(res_v, out_hbm.at[wid])


@functools.cache
def _sc_pos_count():
    return pl.kernel(
        _sc_pos_count_body,
        mesh=plsc.VectorSubcoreMesh(core_axis_name="c", subcore_axis_name="s"),
        out_type=jax.ShapeDtypeStruct((_NW, 16), jnp.float32),
        scratch_types=[
            pltpu.VMEM((_RPW,), jnp.int32),
            pltpu.VMEM((16,), jnp.float32),
        ],
        compiler_params=pltpu.CompilerParams(
            use_tc_tiling_on_sc=False, needs_layout_passes=False
        ),
    )


def _rare_ohem_branch(ops):
    pred, label, pos_num, neg_count, neg_sum = ops
    pred_value = jnp.max(pred[:, 1:], axis=1)
    is_neg = label == 0
    padded = jnp.where(is_neg, -pred_value, jnp.inf)
    sorted_neg_score = jnp.sort(padded)
    raw_idx = neg_sum - 1
    idx = jnp.where(raw_idx >= 0, raw_idx, neg_count + raw_idx)
    idx = jnp.clip(idx, 0, padded.shape[0] - 1)
    threshold = -sorted_neg_score[idx]
    mask = (pred_value >= threshold) | (label != 0)
    masked_label = jnp.where(mask, label, _IGNORE)
    logp = jax.nn.log_softmax(pred, axis=1)
    valid = masked_label != _IGNORE
    safe = jnp.where(valid, masked_label, 0)
    nll = -jnp.take_along_axis(logp, safe[:, None], axis=1)[:, 0]
    denom = jnp.maximum(jnp.sum(valid), 1).astype(pred.dtype)
    return jnp.sum(jnp.where(valid, nll, 0.0)) / denom


def kernel(pred, label):
    pos_parts = _sc_pos_count()(label)
    pred_t = pred.T
    sum_lse, sum_picked = _ce_pass(pred_t, label)
    pos_num = jnp.sum(pos_parts).astype(jnp.int32)
    neg_count = _N - pos_num
    neg_sum = pos_num * _FACTOR
    common = (sum_lse - sum_picked) / jnp.float32(_N)
    return lax.cond(
        neg_count > neg_sum,
        _rare_ohem_branch,
        lambda ops: common,
        (pred, label, pos_num, neg_count, neg_sum),
    )
